# Initial kernel scaffold; baseline (speedup 1.0000x reference)
import jax
import jax.numpy as jnp
from jax.experimental import pallas as pl


def kernel(x, index_uL, index_vL, attr_uL, batch_ids, W_atom, b_atom, We0, be0, WuL0, WvL0, gamma0, beta0, We1, be1, WuL1, WvL1, gamma1, beta1, We2, be2, WuL2, WvL2, gamma2, beta2, W_pool, b_pool):
    raise NotImplementedError("write your pallas kernel here")


# speedup vs baseline: 1.0000x; 1.0000x over previous
import jax
import jax.numpy as jnp
from jax.experimental import pallas as pl
from jax.experimental.pallas import tpu as pltpu


def _final_mm_body(pooled_ref, w_ref, b_ref, out_ref):
    out_ref[...] = (
        jnp.dot(pooled_ref[...], w_ref[...], preferred_element_type=jnp.float32)
        + b_ref[...]
    )


def _norm_relu(s_, gamma, beta):
    mu = jnp.mean(s_, axis=-1, keepdims=True)
    var = jnp.var(s_, axis=-1, keepdims=True)
    return jax.nn.relu((s_ - mu) * jax.lax.rsqrt(var + 1e-5) * gamma + beta)


def _layer(x, index_uL, index_vL, attr_uL, We, be, WuL, WvL, gamma, beta):
    dst_u, src_u = index_uL[0], index_uL[1]
    dst_v, src_v = index_vL[0], index_vL[1]
    e = attr_uL @ We + be
    msg_u = jax.nn.relu(jnp.take(x, src_u, axis=0) + e)
    agg_u = jax.ops.segment_sum(msg_u, dst_u, num_segments=x.shape[0]) @ WuL
    msg_v = jnp.take(x, src_v, axis=0)
    agg_v = jax.ops.segment_sum(msg_v, dst_v, num_segments=x.shape[0]) @ WvL
    h = _norm_relu(agg_u + agg_v, gamma, beta)
    return h + x


def kernel(x, index_uL, index_vL, attr_uL, batch_ids, W_atom, b_atom, We0, be0, WuL0, WvL0, gamma0, beta0, We1, be1, WuL1, WvL1, gamma1, beta1, We2, be2, WuL2, WvL2, gamma2, beta2, W_pool, b_pool):
    x = x @ W_atom + b_atom
    layer_params = [(We0, be0, WuL0, WvL0, gamma0, beta0),
                    (We1, be1, WuL1, WvL1, gamma1, beta1),
                    (We2, be2, WuL2, WvL2, gamma2, beta2)]
    for (We, be, WuL, WvL, gamma, beta) in layer_params:
        x = _layer(x, index_uL, index_vL, attr_uL, We, be, WuL, WvL, gamma, beta)
    pooled = jax.ops.segment_sum(x, batch_ids, num_segments=256)
    out = pl.pallas_call(
        _final_mm_body,
        out_shape=jax.ShapeDtypeStruct((256, 128), jnp.float32),
    )(pooled, W_pool, b_pool)
    return out


# speedup vs baseline: 1.5333x; 1.5333x over previous
import functools

import jax
import jax.numpy as jnp
from jax import lax
from jax.experimental import pallas as pl
from jax.experimental.pallas import tpu as pltpu
from jax.experimental.pallas import tpu_sc as plsc

N = 10000
E = 320000
D = 128
H = 64
DE = 4
G = 256
NS = 16
CH = 128
NCHUNK = 157
EPS = CH * NCHUNK
EP = EPS * NS
NPAD = 10112
ZR = NPAD // NS

R = 2000
TCGRID = 5


def _sc_layer_body(xlo, xhi, srcu, dstu, srcv, dstv, attr,
                   welo, wehi, belo, behi, zeros,
                   aggu_lo, aggu_hi, aggv_lo, aggv_hi,
                   we_v, be_v, srcbuf, dstbuf, attrbuf, rowsbuf,
                   accu, accv, sem):
    c = lax.axis_index("c")
    s = lax.axis_index("s")

    def run(x_ref, we_ref, be_ref, out_u, out_v):
        pltpu.sync_copy(we_ref, we_v)
        pltpu.sync_copy(be_ref, be_v)
        pltpu.sync_copy(zeros.at[pl.ds(s * ZR, ZR)], accu.at[pl.ds(s * ZR, ZR)])
        pltpu.sync_copy(zeros.at[pl.ds(s * ZR, ZR)], accv.at[pl.ds(s * ZR, ZR)])
        plsc.subcore_barrier()

        ebase = s * EPS

        def u_chunk(j, carry):
            base = ebase + j * CH
            pltpu.sync_copy(srcu.at[pl.ds(base, CH)], srcbuf)
            pltpu.sync_copy(dstu.at[pl.ds(base, CH)], dstbuf)
            pltpu.sync_copy(attr.at[:, pl.ds(base, CH)], attrbuf)
            pltpu.async_copy(x_ref.at[srcbuf], rowsbuf, sem).wait()

            def group(gg, carry2):
                avs = [attrbuf[k, pl.ds(gg * 16, 16)] for k in range(DE)]
                for t in range(16):
                    i = gg * 16 + t
                    for v in range(H // 16):
                        sl = pl.ds(v * 16, 16)
                        e = (be_v[sl]
                             + avs[0][t] * we_v[0, sl] + avs[1][t] * we_v[1, sl]
                             + avs[2][t] * we_v[2, sl] + avs[3][t] * we_v[3, sl])
                        rowsbuf[i, sl] = jnp.maximum(rowsbuf[i, sl] + e, 0.0)
                return carry2

            lax.fori_loop(0, CH // 16, group, 0)
            pltpu.sync_copy(rowsbuf, accu.at[dstbuf], add=True)
            return carry

        lax.fori_loop(0, NCHUNK, u_chunk, 0)

        def v_chunk(j, carry):
            base = ebase + j * CH
            pltpu.sync_copy(srcv.at[pl.ds(base, CH)], srcbuf)
            pltpu.sync_copy(dstv.at[pl.ds(base, CH)], dstbuf)
            pltpu.async_copy(x_ref.at[srcbuf], rowsbuf, sem).wait()
            pltpu.sync_copy(rowsbuf, accv.at[dstbuf], add=True)
            return carry

        lax.fori_loop(0, NCHUNK, v_chunk, 0)

        plsc.subcore_barrier()
        pltpu.sync_copy(accu.at[pl.ds(s * ZR, ZR)], out_u.at[pl.ds(s * ZR, ZR)])
        pltpu.sync_copy(accv.at[pl.ds(s * ZR, ZR)], out_v.at[pl.ds(s * ZR, ZR)])

    @pl.when(c == 0)
    def _():
        run(xlo, welo, belo, aggu_lo, aggv_lo)

    @pl.when(c == 1)
    def _():
        run(xhi, wehi, behi, aggu_hi, aggv_hi)


def _make_sc_layer():
    mesh = plsc.VectorSubcoreMesh(core_axis_name="c", subcore_axis_name="s")
    f32 = jnp.float32
    return pl.kernel(
        _sc_layer_body,
        out_type=[jax.ShapeDtypeStruct((NPAD, H), f32) for _ in range(4)],
        mesh=mesh,
        scratch_types=[
            pltpu.VMEM((DE, H), f32),
            pltpu.VMEM((H,), f32),
            pltpu.VMEM((CH,), jnp.int32),
            pltpu.VMEM((CH,), jnp.int32),
            pltpu.VMEM((DE, CH), f32),
            pltpu.VMEM((CH, H), f32),
            pltpu.VMEM_SHARED((NPAD, H), f32),
            pltpu.VMEM_SHARED((NPAD, H), f32),
            pltpu.SemaphoreType.DMA,
        ],
        compiler_params=pltpu.CompilerParams(use_tc_tiling_on_sc=False),
    )


def _atom_body(x_ref, w_ref, b_ref, lo_ref, hi_ref):
    y = jnp.dot(x_ref[...], w_ref[...], preferred_element_type=jnp.float32)
    y = y + b_ref[...]
    lo_ref[...] = y[:, :H]
    hi_ref[...] = y[:, H:]


def _atom_call(x, W_atom, b_atom):
    return pl.pallas_call(
        _atom_body,
        grid=(TCGRID,),
        in_specs=[
            pl.BlockSpec((R, D), lambda i: (i, 0)),
            pl.BlockSpec((D, D), lambda i: (0, 0)),
            pl.BlockSpec((1, D), lambda i: (0, 0)),
        ],
        out_specs=[
            pl.BlockSpec((R, H), lambda i: (i, 0)),
            pl.BlockSpec((R, H), lambda i: (i, 0)),
        ],
        out_shape=[jax.ShapeDtypeStruct((N, H), jnp.float32)] * 2,
    )(x, W_atom, b_atom)


def _tc_layer_body(aul_ref, auh_ref, avl_ref, avh_ref, xl_ref, xh_ref,
                   wu_ref, wv_ref, g_ref, b_ref, lo_ref, hi_ref):
    wu = wu_ref[...]
    wv = wv_ref[...]
    s = (jnp.dot(aul_ref[...], wu[:H, :], preferred_element_type=jnp.float32)
         + jnp.dot(auh_ref[...], wu[H:, :], preferred_element_type=jnp.float32)
         + jnp.dot(avl_ref[...], wv[:H, :], preferred_element_type=jnp.float32)
         + jnp.dot(avh_ref[...], wv[H:, :], preferred_element_type=jnp.float32))
    mu = jnp.mean(s, axis=-1, keepdims=True)
    d = s - mu
    var = jnp.mean(d * d, axis=-1, keepdims=True)
    h = jax.nn.relu(d * lax.rsqrt(var + 1e-5) * g_ref[...] + b_ref[...])
    lo_ref[...] = h[:, :H] + xl_ref[...]
    hi_ref[...] = h[:, H:] + xh_ref[...]


def _tc_layer_call(aul, auh, avl, avh, xl, xh, WuL, WvL, gamma, beta):
    half = pl.BlockSpec((R, H), lambda i: (i, 0))
    full = pl.BlockSpec((D, D), lambda i: (0, 0))
    vec = pl.BlockSpec((1, D), lambda i: (0, 0))
    return pl.pallas_call(
        _tc_layer_body,
        grid=(TCGRID,),
        in_specs=[half, half, half, half, half, half, full, full, vec, vec],
        out_specs=[half, half],
        out_shape=[jax.ShapeDtypeStruct((N, H), jnp.float32)] * 2,
    )(aul, auh, avl, avh, xl, xh, WuL, WvL, gamma, beta)


def _pool_body(xl_ref, xh_ref, ids_ref, w_ref, b_ref, out_ref):
    i = pl.program_id(0)
    w = w_ref[...]
    y = (jnp.dot(xl_ref[...], w[:H, :], preferred_element_type=jnp.float32)
         + jnp.dot(xh_ref[...], w[H:, :], preferred_element_type=jnp.float32))
    ids = ids_ref[...].reshape(1, R)
    onehot = (lax.broadcasted_iota(jnp.int32, (G, R), 0) == ids).astype(jnp.float32)
    part = jnp.dot(onehot, y, preferred_element_type=jnp.float32)

    @pl.when(i == 0)
    def _():
        out_ref[...] = part + jnp.broadcast_to(b_ref[...], (G, D))

    @pl.when(i > 0)
    def _():
        out_ref[...] = out_ref[...] + part


def _pool_call(xl, xh, ids3, W_pool, b_pool):
    return pl.pallas_call(
        _pool_body,
        grid=(TCGRID,),
        in_specs=[
            pl.BlockSpec((R, H), lambda i: (i, 0)),
            pl.BlockSpec((R, H), lambda i: (i, 0)),
            pl.BlockSpec((1, 1, R), lambda i: (i, 0, 0)),
            pl.BlockSpec((D, D), lambda i: (0, 0)),
            pl.BlockSpec((1, D), lambda i: (0, 0)),
        ],
        out_specs=pl.BlockSpec((G, D), lambda i: (0, 0)),
        out_shape=jax.ShapeDtypeStruct((G, D), jnp.float32),
    )(xl, xh, ids3, W_pool, b_pool)


def kernel(x, index_uL, index_vL, attr_uL, batch_ids, W_atom, b_atom,
           We0, be0, WuL0, WvL0, gamma0, beta0,
           We1, be1, WuL1, WvL1, gamma1, beta1,
           We2, be2, WuL2, WvL2, gamma2, beta2,
           W_pool, b_pool):
    i32 = jnp.int32
    f32 = jnp.float32
    pad = EP - E

    def pad_idx(a, val):
        return jnp.concatenate([a, jnp.full((pad,), val, i32)])

    dstu = pad_idx(index_uL[0], N)
    srcu = pad_idx(index_uL[1], 0)
    dstv = pad_idx(index_vL[0], N)
    srcv = pad_idx(index_vL[1], 0)
    attr = jnp.concatenate([attr_uL, jnp.zeros((pad, DE), f32)]).T.copy()
    zeros = jnp.zeros((NPAD, H), f32)
    ids3 = batch_ids.reshape(TCGRID, 1, R)

    xl, xh = _atom_call(x, W_atom, b_atom.reshape(1, D))

    sc_layer = _make_sc_layer()
    for (We, be, WuL, WvL, gamma, beta) in (
            (We0, be0, WuL0, WvL0, gamma0, beta0),
            (We1, be1, WuL1, WvL1, gamma1, beta1),
            (We2, be2, WuL2, WvL2, gamma2, beta2)):
        aul, auh, avl, avh = sc_layer(
            xl, xh, srcu, dstu, srcv, dstv, attr,
            We[:, :H], We[:, H:], be[:H], be[H:], zeros)
        xl, xh = _tc_layer_call(aul, auh, avl, avh, xl, xh, WuL, WvL,
                                gamma.reshape(1, D), beta.reshape(1, D))

    return _pool_call(xl, xh, ids3, W_pool, b_pool.reshape(1, D))


# speedup vs baseline: 1.8049x; 1.1771x over previous
import functools

import jax
import jax.numpy as jnp
from jax import lax
from jax.experimental import pallas as pl
from jax.experimental.pallas import tpu as pltpu
from jax.experimental.pallas import tpu_sc as plsc

N = 10000
E = 320000
D = 128
H = 64
DE = 4
G = 256
NS = 16
CH = 128
NBUF = 4
NCHUNK = 160
EPS = CH * NCHUNK
EP = EPS * NS
NPAD = 10112
ZR = NPAD // NS

R = 2000
TCGRID = 5


def _sc_layer_body(xlo, xhi, srcu, dstu, srcv, dstv, attr,
                   welo, wehi, belo, behi, zeros,
                   aggu_lo, aggu_hi, aggv_lo, aggv_hi,
                   we_v, be_v, srcbuf, dstbuf, attrbuf, rowsbuf,
                   acc, semi, semg, sems):
    c = lax.axis_index("c")
    s = lax.axis_index("s")

    def run(x_ref, we_ref, be_ref, out_u, out_v):
        zsl = pl.ds(s * ZR, ZR)
        pltpu.sync_copy(we_ref, we_v)
        pltpu.sync_copy(be_ref, be_v)
        pltpu.sync_copy(zeros.at[zsl], acc.at[zsl])
        plsc.subcore_barrier()

        ebase = s * EPS

        def compute_chunk(b):
            def group(gg, carry2):
                avs = [attrbuf[b, k, pl.ds(gg * 16, 16)] for k in range(DE)]
                for t in range(16):
                    i = gg * 16 + t
                    for v in range(H // 16):
                        sl = pl.ds(v * 16, 16)
                        e = (be_v[sl]
                             + avs[0][t] * we_v[0, sl] + avs[1][t] * we_v[1, sl]
                             + avs[2][t] * we_v[2, sl] + avs[3][t] * we_v[3, sl])
                        rowsbuf[b, i, sl] = jnp.maximum(rowsbuf[b, i, sl] + e, 0.0)
                return carry2

            lax.fori_loop(0, CH // 16, group, 0)

        def side(src_idx, dst_idx, is_u):
            def outer(og, carry):
                jbase = ebase + og * (NBUF * CH)
                idescs = []
                for b in range(NBUF):
                    base = jbase + b * CH
                    ds_ = [pltpu.async_copy(src_idx.at[pl.ds(base, CH)],
                                            srcbuf.at[b], semi.at[b]),
                           pltpu.async_copy(dst_idx.at[pl.ds(base, CH)],
                                            dstbuf.at[b], semi.at[b])]
                    if is_u:
                        ds_.append(pltpu.async_copy(attr.at[:, pl.ds(base, CH)],
                                                    attrbuf.at[b], semi.at[b]))
                    idescs.append(ds_)
                gdescs = []
                for b in range(NBUF):
                    for d in idescs[b]:
                        d.wait()
                    gdescs.append(pltpu.async_copy(x_ref.at[srcbuf.at[b]],
                                                   rowsbuf.at[b], semg.at[b]))
                sdescs = []
                for b in range(NBUF):
                    gdescs[b].wait()
                    if is_u:
                        compute_chunk(b)
                    sdescs.append(pltpu.async_copy(rowsbuf.at[b],
                                                   acc.at[dstbuf.at[b]],
                                                   sems.at[b], add=True))
                for b in range(NBUF):
                    sdescs[b].wait()
                return carry

            lax.fori_loop(0, NCHUNK // NBUF, outer, 0)

        side(srcu, dstu, True)
        plsc.subcore_barrier()
        pltpu.sync_copy(acc.at[zsl], out_u.at[zsl])
        pltpu.sync_copy(zeros.at[zsl], acc.at[zsl])
        plsc.subcore_barrier()
        side(srcv, dstv, False)
        plsc.subcore_barrier()
        pltpu.sync_copy(acc.at[zsl], out_v.at[zsl])

    @pl.when(c == 0)
    def _():
        run(xlo, welo, belo, aggu_lo, aggv_lo)

    @pl.when(c == 1)
    def _():
        run(xhi, wehi, behi, aggu_hi, aggv_hi)


def _make_sc_layer():
    mesh = plsc.VectorSubcoreMesh(core_axis_name="c", subcore_axis_name="s")
    f32 = jnp.float32
    return pl.kernel(
        _sc_layer_body,
        out_type=[jax.ShapeDtypeStruct((NPAD, H), f32) for _ in range(4)],
        mesh=mesh,
        scratch_types=[
            pltpu.VMEM((DE, H), f32),
            pltpu.VMEM((H,), f32),
            pltpu.VMEM((NBUF, CH), jnp.int32),
            pltpu.VMEM((NBUF, CH), jnp.int32),
            pltpu.VMEM((NBUF, DE, CH), f32),
            pltpu.VMEM((NBUF, CH, H), f32),
            pltpu.VMEM_SHARED((NPAD, H), f32),
            pltpu.SemaphoreType.DMA((NBUF,)),
            pltpu.SemaphoreType.DMA((NBUF,)),
            pltpu.SemaphoreType.DMA((NBUF,)),
        ],
        compiler_params=pltpu.CompilerParams(use_tc_tiling_on_sc=False),
    )


def _atom_body(x_ref, w_ref, b_ref, lo_ref, hi_ref):
    y = jnp.dot(x_ref[...], w_ref[...], preferred_element_type=jnp.float32)
    y = y + b_ref[...]
    lo_ref[...] = y[:, :H]
    hi_ref[...] = y[:, H:]


def _atom_call(x, W_atom, b_atom):
    return pl.pallas_call(
        _atom_body,
        grid=(TCGRID,),
        in_specs=[
            pl.BlockSpec((R, D), lambda i: (i, 0)),
            pl.BlockSpec((D, D), lambda i: (0, 0)),
            pl.BlockSpec((1, D), lambda i: (0, 0)),
        ],
        out_specs=[
            pl.BlockSpec((R, H), lambda i: (i, 0)),
            pl.BlockSpec((R, H), lambda i: (i, 0)),
        ],
        out_shape=[jax.ShapeDtypeStruct((N, H), jnp.float32)] * 2,
    )(x, W_atom, b_atom)


def _tc_layer_body(aul_ref, auh_ref, avl_ref, avh_ref, xl_ref, xh_ref,
                   wu_ref, wv_ref, g_ref, b_ref, lo_ref, hi_ref):
    wu = wu_ref[...]
    wv = wv_ref[...]
    s = (jnp.dot(aul_ref[...], wu[:H, :], preferred_element_type=jnp.float32)
         + jnp.dot(auh_ref[...], wu[H:, :], preferred_element_type=jnp.float32)
         + jnp.dot(avl_ref[...], wv[:H, :], preferred_element_type=jnp.float32)
         + jnp.dot(avh_ref[...], wv[H:, :], preferred_element_type=jnp.float32))
    mu = jnp.mean(s, axis=-1, keepdims=True)
    d = s - mu
    var = jnp.mean(d * d, axis=-1, keepdims=True)
    h = jax.nn.relu(d * lax.rsqrt(var + 1e-5) * g_ref[...] + b_ref[...])
    lo_ref[...] = h[:, :H] + xl_ref[...]
    hi_ref[...] = h[:, H:] + xh_ref[...]


def _tc_layer_call(aul, auh, avl, avh, xl, xh, WuL, WvL, gamma, beta):
    half = pl.BlockSpec((R, H), lambda i: (i, 0))
    full = pl.BlockSpec((D, D), lambda i: (0, 0))
    vec = pl.BlockSpec((1, D), lambda i: (0, 0))
    return pl.pallas_call(
        _tc_layer_body,
        grid=(TCGRID,),
        in_specs=[half, half, half, half, half, half, full, full, vec, vec],
        out_specs=[half, half],
        out_shape=[jax.ShapeDtypeStruct((N, H), jnp.float32)] * 2,
    )(aul, auh, avl, avh, xl, xh, WuL, WvL, gamma, beta)


def _pool_body(xl_ref, xh_ref, ids_ref, w_ref, b_ref, out_ref):
    i = pl.program_id(0)
    w = w_ref[...]
    y = (jnp.dot(xl_ref[...], w[:H, :], preferred_element_type=jnp.float32)
         + jnp.dot(xh_ref[...], w[H:, :], preferred_element_type=jnp.float32))
    ids = ids_ref[...].reshape(1, R)
    onehot = (lax.broadcasted_iota(jnp.int32, (G, R), 0) == ids).astype(jnp.float32)
    part = jnp.dot(onehot, y, preferred_element_type=jnp.float32)

    @pl.when(i == 0)
    def _():
        out_ref[...] = part + jnp.broadcast_to(b_ref[...], (G, D))

    @pl.when(i > 0)
    def _():
        out_ref[...] = out_ref[...] + part


def _pool_call(xl, xh, ids3, W_pool, b_pool):
    return pl.pallas_call(
        _pool_body,
        grid=(TCGRID,),
        in_specs=[
            pl.BlockSpec((R, H), lambda i: (i, 0)),
            pl.BlockSpec((R, H), lambda i: (i, 0)),
            pl.BlockSpec((1, 1, R), lambda i: (i, 0, 0)),
            pl.BlockSpec((D, D), lambda i: (0, 0)),
            pl.BlockSpec((1, D), lambda i: (0, 0)),
        ],
        out_specs=pl.BlockSpec((G, D), lambda i: (0, 0)),
        out_shape=jax.ShapeDtypeStruct((G, D), jnp.float32),
    )(xl, xh, ids3, W_pool, b_pool)


def kernel(x, index_uL, index_vL, attr_uL, batch_ids, W_atom, b_atom,
           We0, be0, WuL0, WvL0, gamma0, beta0,
           We1, be1, WuL1, WvL1, gamma1, beta1,
           We2, be2, WuL2, WvL2, gamma2, beta2,
           W_pool, b_pool):
    i32 = jnp.int32
    f32 = jnp.float32
    pad = EP - E

    def pad_idx(a, val):
        return jnp.concatenate([a, jnp.full((pad,), val, i32)])

    dstu = pad_idx(index_uL[0], N)
    srcu = pad_idx(index_uL[1], 0)
    dstv = pad_idx(index_vL[0], N)
    srcv = pad_idx(index_vL[1], 0)
    attr = jnp.concatenate([attr_uL, jnp.zeros((pad, DE), f32)]).T.copy()
    zeros = jnp.zeros((NPAD, H), f32)
    ids3 = batch_ids.reshape(TCGRID, 1, R)

    xl, xh = _atom_call(x, W_atom, b_atom.reshape(1, D))

    sc_layer = _make_sc_layer()
    Wes = jnp.stack([We0, We1, We2])
    bes = jnp.stack([be0, be1, be2])
    Wus = jnp.stack([WuL0, WuL1, WuL2])
    Wvs = jnp.stack([WvL0, WvL1, WvL2])
    gs = jnp.stack([gamma0.reshape(1, D), gamma1.reshape(1, D), gamma2.reshape(1, D)])
    bs = jnp.stack([beta0.reshape(1, D), beta1.reshape(1, D), beta2.reshape(1, D)])

    def step(carry, ws):
        cxl, cxh = carry
        We, be, WuL, WvL, g, b = ws
        aul, auh, avl, avh = sc_layer(
            cxl, cxh, srcu, dstu, srcv, dstv, attr,
            We[:, :H], We[:, H:], be[:H], be[H:], zeros)
        nxl, nxh = _tc_layer_call(aul, auh, avl, avh, cxl, cxh, WuL, WvL, g, b)
        return (nxl, nxh), None

    (xl, xh), _ = lax.scan(step, (xl, xh), (Wes, bes, Wus, Wvs, gs, bs))

    return _pool_call(xl, xh, ids3, W_pool, b_pool.reshape(1, D))


# speedup vs baseline: 2.1668x; 1.2005x over previous
import functools

import jax
import jax.numpy as jnp
from jax import lax
from jax.experimental import pallas as pl
from jax.experimental.pallas import tpu as pltpu
from jax.experimental.pallas import tpu_sc as plsc

N = 10000
E = 320000
D = 128
H = 64
DE = 4
G = 256
NS = 16
CH = 128
NBUF = 2
NCHUNK = 160
EPS = CH * NCHUNK
EP = EPS * NS
NPAD = 10112
ZR = NPAD // NS

R = 2000
TCGRID = 5


def _sc_layer_body(xlo, xhi, srcu, dstu, srcv, dstv, attr,
                   welo, wehi, belo, behi, zeros,
                   aggu_lo, aggu_hi, aggv_lo, aggv_hi,
                   we_v, be_v, srcall, dstall, attrbuf, rowsbuf,
                   acc, semi, semg, sems):
    c = lax.axis_index("c")
    s = lax.axis_index("s")

    def run(x_ref, we_ref, be_ref, out_u, out_v):
        zsl = pl.ds(s * ZR, ZR)
        pltpu.sync_copy(we_ref, we_v)
        pltpu.sync_copy(be_ref, be_v)
        pltpu.sync_copy(zeros.at[zsl], acc.at[zsl])
        plsc.subcore_barrier()

        def compute_chunk(rb, b):
            def group(gg, carry2):
                avs = [attrbuf[b, k, pl.ds(gg * 16, 16)] for k in range(DE)]
                for t in range(16):
                    i = gg * 16 + t
                    for v in range(H // 16):
                        sl = pl.ds(v * 16, 16)
                        e = (be_v[sl]
                             + avs[0][t] * we_v[0, sl] + avs[1][t] * we_v[1, sl]
                             + avs[2][t] * we_v[2, sl] + avs[3][t] * we_v[3, sl])
                        rowsbuf[rb, i, sl] = jnp.maximum(rowsbuf[rb, i, sl] + e, 0.0)
                return carry2

            lax.fori_loop(0, CH // 16, group, 0)

        def wait_rows(sem_slot, rb):
            pltpu.make_async_copy(x_ref.at[pl.ds(0, CH)], rowsbuf.at[rb],
                                  sem_slot).wait()

        def wait_attr(b):
            pltpu.make_async_copy(attr.at[:, pl.ds(0, CH)], attrbuf.at[b],
                                  semi.at[b]).wait()

        def side(src_idx, dst_idx, is_u):
            csl = pl.ds(s * NCHUNK, NCHUNK)
            pltpu.sync_copy(src_idx.at[csl], srcall)
            pltpu.sync_copy(dst_idx.at[csl], dstall)
            abase = s * EPS

            for b in range(NBUF):
                if is_u:
                    pltpu.async_copy(attr.at[:, pl.ds(abase + b * CH, CH)],
                                     attrbuf.at[b], semi.at[b])
                pltpu.async_copy(x_ref.at[srcall.at[b]], rowsbuf.at[b],
                                 semg.at[b])

            def outer(og2, carry):
                for par in range(2):
                    og = og2 * 2 + par
                    for b in range(NBUF):
                        j = og * NBUF + b
                        rb = par * NBUF + b
                        ob = (1 - par) * NBUF + b
                        wait_rows(semg.at[b], rb)
                        if is_u:
                            wait_attr(b)
                            compute_chunk(rb, b)
                        pltpu.async_copy(rowsbuf.at[rb], acc.at[dstall.at[j]],
                                         sems.at[b], add=True)

                        @pl.when(j + NBUF < NCHUNK)
                        def _():
                            if is_u:
                                pltpu.async_copy(
                                    attr.at[:, pl.ds(abase + (j + NBUF) * CH, CH)],
                                    attrbuf.at[b], semi.at[b])

                            @pl.when(j >= NBUF)
                            def _():
                                wait_rows(sems.at[b], ob)

                            pltpu.async_copy(x_ref.at[srcall.at[j + NBUF]],
                                             rowsbuf.at[ob], semg.at[b])
                return carry

            lax.fori_loop(0, NCHUNK // (2 * NBUF), outer, 0)

            for par in range(2):
                for b in range(NBUF):
                    wait_rows(sems.at[b], par * NBUF + b)

        side(srcu, dstu, True)
        plsc.subcore_barrier()
        pltpu.sync_copy(acc.at[zsl], out_u.at[zsl])
        pltpu.sync_copy(zeros.at[zsl], acc.at[zsl])
        plsc.subcore_barrier()
        side(srcv, dstv, False)
        plsc.subcore_barrier()
        pltpu.sync_copy(acc.at[zsl], out_v.at[zsl])

    @pl.when(c == 0)
    def _():
        run(xlo, welo, belo, aggu_lo, aggv_lo)

    @pl.when(c == 1)
    def _():
        run(xhi, wehi, behi, aggu_hi, aggv_hi)


def _make_sc_layer():
    mesh = plsc.VectorSubcoreMesh(core_axis_name="c", subcore_axis_name="s")
    f32 = jnp.float32
    return pl.kernel(
        _sc_layer_body,
        out_type=[jax.ShapeDtypeStruct((NPAD, H), f32) for _ in range(4)],
        mesh=mesh,
        scratch_types=[
            pltpu.VMEM((DE, H), f32),
            pltpu.VMEM((H,), f32),
            pltpu.VMEM((NCHUNK, CH), jnp.int32),
            pltpu.VMEM((NCHUNK, CH), jnp.int32),
            pltpu.VMEM((NBUF, DE, CH), f32),
            pltpu.VMEM((2 * NBUF, CH, H), f32),
            pltpu.VMEM_SHARED((NPAD, H), f32),
            pltpu.SemaphoreType.DMA((NBUF,)),
            pltpu.SemaphoreType.DMA((NBUF,)),
            pltpu.SemaphoreType.DMA((NBUF,)),
        ],
        compiler_params=pltpu.CompilerParams(use_tc_tiling_on_sc=False),
    )


def _atom_body(x_ref, w_ref, b_ref, lo_ref, hi_ref):
    y = jnp.dot(x_ref[...], w_ref[...], preferred_element_type=jnp.float32)
    y = y + b_ref[...]
    lo_ref[...] = y[:, :H]
    hi_ref[...] = y[:, H:]


def _atom_call(x, W_atom, b_atom):
    return pl.pallas_call(
        _atom_body,
        grid=(TCGRID,),
        in_specs=[
            pl.BlockSpec((R, D), lambda i: (i, 0)),
            pl.BlockSpec((D, D), lambda i: (0, 0)),
            pl.BlockSpec((1, D), lambda i: (0, 0)),
        ],
        out_specs=[
            pl.BlockSpec((R, H), lambda i: (i, 0)),
            pl.BlockSpec((R, H), lambda i: (i, 0)),
        ],
        out_shape=[jax.ShapeDtypeStruct((N, H), jnp.float32)] * 2,
    )(x, W_atom, b_atom)


def _tc_layer_body(aul_ref, auh_ref, avl_ref, avh_ref, xl_ref, xh_ref,
                   wu_ref, wv_ref, g_ref, b_ref, lo_ref, hi_ref):
    wu = wu_ref[...]
    wv = wv_ref[...]
    s = (jnp.dot(aul_ref[...], wu[:H, :], preferred_element_type=jnp.float32)
         + jnp.dot(auh_ref[...], wu[H:, :], preferred_element_type=jnp.float32)
         + jnp.dot(avl_ref[...], wv[:H, :], preferred_element_type=jnp.float32)
         + jnp.dot(avh_ref[...], wv[H:, :], preferred_element_type=jnp.float32))
    mu = jnp.mean(s, axis=-1, keepdims=True)
    d = s - mu
    var = jnp.mean(d * d, axis=-1, keepdims=True)
    h = jax.nn.relu(d * lax.rsqrt(var + 1e-5) * g_ref[...] + b_ref[...])
    lo_ref[...] = h[:, :H] + xl_ref[...]
    hi_ref[...] = h[:, H:] + xh_ref[...]


def _tc_layer_call(aul, auh, avl, avh, xl, xh, WuL, WvL, gamma, beta):
    half = pl.BlockSpec((R, H), lambda i: (i, 0))
    full = pl.BlockSpec((D, D), lambda i: (0, 0))
    vec = pl.BlockSpec((1, D), lambda i: (0, 0))
    return pl.pallas_call(
        _tc_layer_body,
        grid=(TCGRID,),
        in_specs=[half, half, half, half, half, half, full, full, vec, vec],
        out_specs=[half, half],
        out_shape=[jax.ShapeDtypeStruct((N, H), jnp.float32)] * 2,
    )(aul, auh, avl, avh, xl, xh, WuL, WvL, gamma, beta)


def _pool_body(xl_ref, xh_ref, ids_ref, w_ref, b_ref, out_ref):
    i = pl.program_id(0)
    w = w_ref[...]
    y = (jnp.dot(xl_ref[...], w[:H, :], preferred_element_type=jnp.float32)
         + jnp.dot(xh_ref[...], w[H:, :], preferred_element_type=jnp.float32))
    ids = ids_ref[...].reshape(1, R)
    onehot = (lax.broadcasted_iota(jnp.int32, (G, R), 0) == ids).astype(jnp.float32)
    part = jnp.dot(onehot, y, preferred_element_type=jnp.float32)

    @pl.when(i == 0)
    def _():
        out_ref[...] = part + jnp.broadcast_to(b_ref[...], (G, D))

    @pl.when(i > 0)
    def _():
        out_ref[...] = out_ref[...] + part


def _pool_call(xl, xh, ids3, W_pool, b_pool):
    return pl.pallas_call(
        _pool_body,
        grid=(TCGRID,),
        in_specs=[
            pl.BlockSpec((R, H), lambda i: (i, 0)),
            pl.BlockSpec((R, H), lambda i: (i, 0)),
            pl.BlockSpec((1, 1, R), lambda i: (i, 0, 0)),
            pl.BlockSpec((D, D), lambda i: (0, 0)),
            pl.BlockSpec((1, D), lambda i: (0, 0)),
        ],
        out_specs=pl.BlockSpec((G, D), lambda i: (0, 0)),
        out_shape=jax.ShapeDtypeStruct((G, D), jnp.float32),
    )(xl, xh, ids3, W_pool, b_pool)


def kernel(x, index_uL, index_vL, attr_uL, batch_ids, W_atom, b_atom,
           We0, be0, WuL0, WvL0, gamma0, beta0,
           We1, be1, WuL1, WvL1, gamma1, beta1,
           We2, be2, WuL2, WvL2, gamma2, beta2,
           W_pool, b_pool):
    i32 = jnp.int32
    f32 = jnp.float32
    pad = EP - E

    def pad_idx(a, val):
        return jnp.concatenate([a, jnp.full((pad,), val, i32)]).reshape(
            NS * NCHUNK, CH)

    dstu = pad_idx(index_uL[0], N)
    srcu = pad_idx(index_uL[1], 0)
    dstv = pad_idx(index_vL[0], N)
    srcv = pad_idx(index_vL[1], 0)
    attr = jnp.concatenate([attr_uL, jnp.zeros((pad, DE), f32)]).T.copy()
    zeros = jnp.zeros((NPAD, H), f32)
    ids3 = batch_ids.reshape(TCGRID, 1, R)

    xl, xh = _atom_call(x, W_atom, b_atom.reshape(1, D))

    sc_layer = _make_sc_layer()
    Wes = jnp.stack([We0, We1, We2])
    bes = jnp.stack([be0, be1, be2])
    Wus = jnp.stack([WuL0, WuL1, WuL2])
    Wvs = jnp.stack([WvL0, WvL1, WvL2])
    gs = jnp.stack([gamma0.reshape(1, D), gamma1.reshape(1, D), gamma2.reshape(1, D)])
    bs = jnp.stack([beta0.reshape(1, D), beta1.reshape(1, D), beta2.reshape(1, D)])

    def step(carry, ws):
        cxl, cxh = carry
        We, be, WuL, WvL, g, b = ws
        aul, auh, avl, avh = sc_layer(
            cxl, cxh, srcu, dstu, srcv, dstv, attr,
            We[:, :H], We[:, H:], be[:H], be[H:], zeros)
        nxl, nxh = _tc_layer_call(aul, auh, avl, avh, cxl, cxh, WuL, WvL, g, b)
        return (nxl, nxh), None

    (xl, xh), _ = lax.scan(step, (xl, xh), (Wes, bes, Wus, Wvs, gs, bs))

    return _pool_call(xl, xh, ids3, W_pool, b_pool.reshape(1, D))
